# baseline (speedup 1.0000x reference)
import jax
import jax.numpy as jnp
from jax.experimental import pallas as pl

_ANCHORS = ((10.0, 13.0), (16.0, 30.0), (33.0, 23.0))
_IMG_DIM = 608.0


def _decode_kernel(x_ref, o_ref):
    nG = x_ref.shape[2]
    stride = _IMG_DIM / nG
    gx = jax.lax.broadcasted_iota(jnp.int32, (nG, nG), 1).astype(jnp.float32)
    gy = jax.lax.broadcasted_iota(jnp.int32, (nG, nG), 0).astype(jnp.float32)
    for a, (aw, ah) in enumerate(_ANCHORS):
        tx = x_ref[0, 5 * a + 0]
        ty = x_ref[0, 5 * a + 1]
        tw = x_ref[0, 5 * a + 2]
        th = x_ref[0, 5 * a + 3]
        tc = x_ref[0, 5 * a + 4]
        bx = (jax.nn.sigmoid(tx) + gx) * stride
        by = (jax.nn.sigmoid(ty) + gy) * stride
        bw = jnp.exp(tw) * aw
        bh = jnp.exp(th) * ah
        conf = jax.nn.sigmoid(tc)
        o_ref[0, a] = jnp.stack([bx, by, bw, bh, conf], axis=-1)


def kernel(x):
    B, C, nG, _ = x.shape
    A = len(_ANCHORS)
    out = pl.pallas_call(
        _decode_kernel,
        grid=(B,),
        in_specs=[pl.BlockSpec((1, C, nG, nG), lambda b: (b, 0, 0, 0))],
        out_specs=pl.BlockSpec((1, A, nG, nG, 5), lambda b: (b, 0, 0, 0, 0)),
        out_shape=jax.ShapeDtypeStruct((B, A, nG, nG, 5), jnp.float32),
    )(x)
    return out.reshape(B, A * nG * nG, 5)


# speedup vs baseline: 2.7555x; 2.7555x over previous
import jax
import jax.numpy as jnp
from jax.experimental import pallas as pl

_ANCHORS = ((10.0, 13.0), (16.0, 30.0), (33.0, 23.0))
_IMG_DIM = 608.0


def _decode_kernel(x_ref, o_ref):
    hw = x_ref.shape[2]
    ng = int(round(hw ** 0.5))
    stride = _IMG_DIM / ng
    p = jax.lax.broadcasted_iota(jnp.int32, (1, hw), 1)
    gx = (p % ng).astype(jnp.float32)
    gy = (p // ng).astype(jnp.float32)
    for a, (aw, ah) in enumerate(_ANCHORS):
        tx = x_ref[0, 5 * a + 0:5 * a + 1, :]
        ty = x_ref[0, 5 * a + 1:5 * a + 2, :]
        tw = x_ref[0, 5 * a + 2:5 * a + 3, :]
        th = x_ref[0, 5 * a + 3:5 * a + 4, :]
        tc = x_ref[0, 5 * a + 4:5 * a + 5, :]
        bx = (jax.nn.sigmoid(tx) + gx) * stride
        by = (jax.nn.sigmoid(ty) + gy) * stride
        bw = jnp.exp(tw) * aw
        bh = jnp.exp(th) * ah
        conf = jax.nn.sigmoid(tc)
        y = jnp.concatenate([bx, by, bw, bh, conf], axis=0)
        o_ref[0, a] = y.T


def kernel(x):
    B, C, nG, _ = x.shape
    A = len(_ANCHORS)
    HW = nG * nG
    xf = x.reshape(B, C, HW)
    out = pl.pallas_call(
        _decode_kernel,
        grid=(B,),
        in_specs=[pl.BlockSpec((1, C, HW), lambda b: (b, 0, 0))],
        out_specs=pl.BlockSpec((1, A, HW, 5), lambda b: (b, 0, 0, 0)),
        out_shape=jax.ShapeDtypeStruct((B, A, HW, 5), jnp.float32),
    )(xf)
    return out.reshape(B, A * HW, 5)


# speedup vs baseline: 2.7597x; 1.0015x over previous
import jax
import jax.numpy as jnp
from jax.experimental import pallas as pl
from jax.experimental.pallas import tpu as pltpu

_ANCHORS = ((10.0, 13.0), (16.0, 30.0), (33.0, 23.0))
_IMG_DIM = 608.0


def _decode_kernel(x_ref, o_ref):
    hw = x_ref.shape[2]
    ng = int(round(hw ** 0.5))
    stride = _IMG_DIM / ng
    p = jax.lax.broadcasted_iota(jnp.int32, (1, hw), 1)
    gx = (p % ng).astype(jnp.float32)
    gy = (p // ng).astype(jnp.float32)
    for a, (aw, ah) in enumerate(_ANCHORS):
        tx = x_ref[0, 5 * a + 0:5 * a + 1, :]
        ty = x_ref[0, 5 * a + 1:5 * a + 2, :]
        tw = x_ref[0, 5 * a + 2:5 * a + 3, :]
        th = x_ref[0, 5 * a + 3:5 * a + 4, :]
        tc = x_ref[0, 5 * a + 4:5 * a + 5, :]
        bx = (jax.nn.sigmoid(tx) + gx) * stride
        by = (jax.nn.sigmoid(ty) + gy) * stride
        bw = jnp.exp(tw) * aw
        bh = jnp.exp(th) * ah
        conf = jax.nn.sigmoid(tc)
        y = jnp.concatenate([bx, by, bw, bh, conf], axis=0)
        o_ref[0, a] = y.T


def kernel(x):
    B, C, nG, _ = x.shape
    A = len(_ANCHORS)
    HW = nG * nG
    xf = x.reshape(B, C, HW)
    out = pl.pallas_call(
        _decode_kernel,
        grid=(B,),
        in_specs=[pl.BlockSpec((1, C, HW), lambda b: (b, 0, 0))],
        out_specs=pl.BlockSpec((1, A, HW, 5), lambda b: (b, 0, 0, 0)),
        out_shape=jax.ShapeDtypeStruct((B, A, HW, 5), jnp.float32),
        compiler_params=pltpu.CompilerParams(
            dimension_semantics=("parallel",)),
    )(xf)
    return out.reshape(B, A * HW, 5)
